# Initial kernel scaffold; baseline (speedup 1.0000x reference)
import jax
import jax.numpy as jnp
from jax.experimental import pallas as pl


def kernel(node_features, edge_index, Win, b_in, W0, a0, W1, a1, W2, a2, Wpool, b_pool):
    raise NotImplementedError("write your pallas kernel here")


# speedup vs baseline: 20.3445x; 20.3445x over previous
import functools

import jax
import jax.numpy as jnp
from jax import lax
from jax.experimental import pallas as pl
from jax.experimental.pallas import tpu as pltpu
from jax.experimental.pallas import tpu_sc as plsc

N = 50000
E = 800000
DIN = 25
HID = 64
NH = 4

NCORES = 2
NSUB = 16
NW = NCORES * NSUB

HALF = N // 2
SLAB = 1568
HPAD = NSUB * SLAB
TAIL = HALF - (NSUB - 1) * SLAB

C = 64
EPT_MSG = E // NSUB
NCH_MSG = (EPT_MSG + C - 1) // C
EPT_ST = E // NW
NCH_ST = (EPT_ST + C - 1) // C
EPAD = E + 256

_mesh = plsc.VectorSubcoreMesh(core_axis_name="c", subcore_axis_name="s")


def _dense0_body(x_ref, w1_ref, b_ref, w_ref, wsa_ref, h_ref, s_ref):
    x = jnp.dot(x_ref[...], w1_ref[...], preferred_element_type=jnp.float32)
    x = x + b_ref[...]
    h_ref[...] = jnp.dot(x, w_ref[...], preferred_element_type=jnp.float32)
    s_ref[...] = jnp.dot(x, wsa_ref[...], preferred_element_type=jnp.float32)


def _densek_body(x_ref, w_ref, wsa_ref, h_ref, s_ref):
    v = x_ref[...]
    x = jnp.where(v > 0, v, jnp.exp(v) - 1.0)
    h_ref[...] = jnp.dot(x, w_ref[...], preferred_element_type=jnp.float32)
    s_ref[...] = jnp.dot(x, wsa_ref[...], preferred_element_type=jnp.float32)


_DENSE_OUT = dict(
    out_specs=[
        pl.BlockSpec((2000, NH * HID), lambda i: (i, 0)),
        pl.BlockSpec((2000, 16), lambda i: (i, 0)),
    ],
    out_shape=[
        jax.ShapeDtypeStruct((N, NH * HID), jnp.float32),
        jax.ShapeDtypeStruct((N, 16), jnp.float32),
    ],
)


def _dense0(xin, w1, b2, w, wsa):
    blk = 2000
    return pl.pallas_call(
        _dense0_body,
        grid=(N // blk,),
        in_specs=[
            pl.BlockSpec((blk, DIN), lambda i: (i, 0)),
            pl.BlockSpec((DIN, HID), lambda i: (0, 0)),
            pl.BlockSpec((1, HID), lambda i: (0, 0)),
            pl.BlockSpec((HID, NH * HID), lambda i: (0, 0)),
            pl.BlockSpec((HID, 16), lambda i: (0, 0)),
        ],
        **_DENSE_OUT,
    )(xin, w1, b2, w, wsa)


def _densek(xin, w, wsa):
    blk = 2000
    return pl.pallas_call(
        _densek_body,
        grid=(N // blk,),
        in_specs=[
            pl.BlockSpec((blk, HID), lambda i: (i, 0)),
            pl.BlockSpec((HID, NH * HID), lambda i: (0, 0)),
            pl.BlockSpec((HID, 16), lambda i: (0, 0)),
        ],
        **_DENSE_OUT,
    )(xin, w, wsa)


def _pool_body(o_ref, wp_ref, bp_ref, x_ref, he_ref, acc_ref, m_ref, z_ref):
    i = pl.program_id(0)

    @pl.when(i == 0)
    def _():
        m_ref[0, 0] = -1e30
        z_ref[0, 0] = 0.0
        acc_ref[...] = jnp.zeros_like(acc_ref)

    v = o_ref[...]
    x = jnp.where(v > 0, v, jnp.exp(v) - 1.0)
    x_ref[...] = x
    sc = jnp.sum(x * wp_ref[...], axis=1, keepdims=True) + bp_ref[0, 0]
    bm = jnp.max(sc)
    m_old = m_ref[0, 0]
    m_new = jnp.maximum(m_old, bm)
    scale = jnp.exp(m_old - m_new)
    e = jnp.exp(sc - m_new)
    z_new = z_ref[0, 0] * scale + jnp.sum(e)
    z_ref[0, 0] = z_new
    acc = acc_ref[0:1, :] * scale + jnp.sum(x * e, axis=0, keepdims=True)
    acc_ref[0:1, :] = acc
    m_ref[0, 0] = m_new
    he_ref[...] = acc / jnp.maximum(z_new, 1e-30)


def _pool(out2, wpT, bp):
    blk = 2000
    grid = (N // blk,)
    return pl.pallas_call(
        _pool_body,
        grid=grid,
        in_specs=[
            pl.BlockSpec((blk, HID), lambda i: (i, 0)),
            pl.BlockSpec((1, HID), lambda i: (0, 0)),
            pl.BlockSpec((1, 1), lambda i: (0, 0)),
        ],
        out_specs=[
            pl.BlockSpec((blk, HID), lambda i: (i, 0)),
            pl.BlockSpec((1, HID), lambda i: (0, 0)),
        ],
        out_shape=[
            jax.ShapeDtypeStruct((N, HID), jnp.float32),
            jax.ShapeDtypeStruct((1, HID), jnp.float32),
        ],
        scratch_shapes=[
            pltpu.VMEM((1, HID), jnp.float32),
            pltpu.SMEM((1, 1), jnp.float32),
            pltpu.SMEM((1, 1), jnp.float32),
        ],
        compiler_params=pltpu.CompilerParams(
            dimension_semantics=("arbitrary",)),
    )(out2, wpT, bp)


def _take(x, idx):
    dn = lax.GatherDimensionNumbers(
        offset_dims=(), collapsed_slice_dims=(0,), start_index_map=(0,))
    return lax.gather(x, idx[:, None], dn, (1,),
                      mode=lax.GatherScatterMode.PROMISE_IN_BOUNDS)


@functools.partial(
    pl.kernel,
    out_type=jax.ShapeDtypeStruct((NW, 16), jnp.float32),
    mesh=_mesh,
    scratch_types=[
        pltpu.VMEM((C,), jnp.int32),
        pltpu.VMEM((C,), jnp.int32),
        pltpu.VMEM((C, 16), jnp.float32),
        pltpu.VMEM((C, 16), jnp.float32),
        pltpu.VMEM((16,), jnp.float32),
        pltpu.SemaphoreType.DMA,
        pltpu.SemaphoreType.DMA,
    ],
    compiler_params=pltpu.CompilerParams(use_tc_tiling_on_sc=False),
)
def _stats_k(src_hbm, dst_hbm, s_hbm, stats_hbm,
             srcv, dstv, ssv, sdv, statv, sem1, sem2):
    cid = lax.axis_index("c")
    sid = lax.axis_index("s")
    tid = sid * NCORES + cid
    base = tid * EPT_ST
    iota = lax.iota(jnp.int32, 16)
    sh4 = jnp.minimum(iota + NH, 15)
    bk4 = jnp.maximum(iota - NH, 0)

    def chunk(ci, carry):
        off = base + ci * C
        pltpu.sync_copy(src_hbm.at[pl.ds(off, C)], srcv)
        pltpu.sync_copy(dst_hbm.at[pl.ds(off, C)], dstv)
        cp1 = pltpu.async_copy(s_hbm.at[srcv], ssv, sem1)
        cp2 = pltpu.async_copy(s_hbm.at[dstv], sdv, sem2)
        cp1.wait()
        cp2.wait()

        def edge(e, ms):
            m, z = ms
            valid = (ci * C + e) < EPT_ST
            l = ssv[e] + _take(sdv[e], sh4)
            l = jnp.where(l > 0, l, 0.2 * l)
            l = jnp.where(valid, l, -1e30)
            m_new = jnp.maximum(m, l)
            et = jnp.where(valid, jnp.exp(l - m_new), 0.0)
            z = z * jnp.exp(m - m_new) + et
            return (m_new, z)

        return lax.fori_loop(0, C, edge, carry)

    m, z = lax.fori_loop(
        0, NCH_ST, chunk,
        (jnp.full((16,), -1e30, jnp.float32), jnp.zeros((16,), jnp.float32)))

    v = jnp.where(iota < NH, m, jnp.where(iota < 2 * NH, _take(z, bk4), 0.0))
    statv[...] = v
    pltpu.sync_copy(statv, stats_hbm.at[tid])


@functools.partial(
    pl.kernel,
    out_type=jax.ShapeDtypeStruct((N, HID), jnp.float32),
    mesh=_mesh,
    scratch_types=[
        pltpu.VMEM((C,), jnp.int32),
        pltpu.VMEM((C,), jnp.int32),
        pltpu.VMEM((C,), jnp.int32),
        pltpu.VMEM((C, 16), jnp.float32),
        pltpu.VMEM((C, 16), jnp.float32),
        pltpu.VMEM((C, NH * HID), jnp.float32),
        pltpu.VMEM((C, HID), jnp.float32),
        pltpu.VMEM((NW, 16), jnp.float32),
        pltpu.VMEM_SHARED((HPAD, HID), jnp.float32),
        pltpu.SemaphoreType.DMA,
        pltpu.SemaphoreType.DMA,
        pltpu.SemaphoreType.DMA,
    ],
    compiler_params=pltpu.CompilerParams(use_tc_tiling_on_sc=False),
)
def _msg_k(src_hbm, dst_hbm, s_hbm, h_hbm, stats_hbm, zeros_hbm, out_hbm,
           srcv, dstv, sidxv, ssv, sdv, hhv, msgv,
           statv, accsh, sem1, sem2, sem3):
    cid = lax.axis_index("c")
    sid = lax.axis_index("s")
    iota = lax.iota(jnp.int32, 16)
    sh4 = jnp.minimum(iota + NH, 15)

    pltpu.sync_copy(zeros_hbm, accsh.at[pl.ds(sid * SLAB, SLAB)])
    plsc.subcore_barrier()

    pltpu.sync_copy(stats_hbm, statv)
    m16 = jnp.full((16,), -1e30, jnp.float32)
    for t in range(NW):
        m16 = jnp.maximum(m16, statv[t])
    z16 = jnp.zeros((16,), jnp.float32)
    for t in range(NW):
        row = statv[t]
        z16 = z16 + _take(row, sh4) * jnp.exp(row - m16)
    wvec = 1.0 / (NH * z16)

    base = sid * EPT_MSG
    dlo = cid * HALF

    def chunk(ci, _):
        off = base + ci * C
        pltpu.sync_copy(src_hbm.at[pl.ds(off, C)], srcv)
        pltpu.sync_copy(dst_hbm.at[pl.ds(off, C)], dstv)
        cp1 = pltpu.async_copy(s_hbm.at[srcv], ssv, sem1)
        cp2 = pltpu.async_copy(s_hbm.at[dstv], sdv, sem2)
        cp3 = pltpu.async_copy(h_hbm.at[srcv], hhv, sem3)
        cp1.wait()
        cp2.wait()

        for g in range(C // 16):
            rows = iota + g * 16
            valid = (rows + ci * C) < EPT_MSG
            dv = dstv[pl.ds(g * 16, 16)]
            dl = dv - dlo
            inhalf = valid & (dl >= 0) & (dl < HALF)
            sidxv[pl.ds(g * 16, 16)] = jnp.where(inhalf, dl, HALF)

        cp3.wait()

        def edge(e, _):
            l = ssv[e] + _take(sdv[e], sh4)
            l = jnp.where(l > 0, l, 0.2 * l)
            w16 = jnp.exp(l - m16) * wvec
            wb = [_take(w16, jnp.full((16,), h, jnp.int32)) for h in range(NH)]
            for vblk in range(HID // 16):
                acc = wb[0] * hhv[e, pl.ds(vblk * 16, 16)]
                for h in range(1, NH):
                    acc = acc + wb[h] * hhv[e, pl.ds(h * HID + vblk * 16, 16)]
                msgv[e, pl.ds(vblk * 16, 16)] = acc
            return 0

        lax.fori_loop(0, C, edge, 0)
        pltpu.sync_copy(msgv, accsh.at[sidxv], add=True)
        return 0

    lax.fori_loop(0, NCH_MSG, chunk, 0)
    plsc.subcore_barrier()

    @pl.when(sid < NSUB - 1)
    def _():
        pltpu.sync_copy(accsh.at[pl.ds(sid * SLAB, SLAB)],
                        out_hbm.at[pl.ds(dlo + sid * SLAB, SLAB)])

    @pl.when(sid == NSUB - 1)
    def _():
        pltpu.sync_copy(accsh.at[pl.ds((NSUB - 1) * SLAB, TAIL)],
                        out_hbm.at[pl.ds(dlo + (NSUB - 1) * SLAB, TAIL)])


def _build_wsa(w, a):
    a_src = a[:, :HID]
    a_dst = a[:, HID:]
    eye = jnp.eye(NH, dtype=jnp.float32)
    m1 = (eye[:, None, :] * a_src[:, :, None]).reshape(NH * HID, NH)
    m2 = (eye[:, None, :] * a_dst[:, :, None]).reshape(NH * HID, NH)
    wsa = w @ jnp.concatenate([m1, m2], axis=1)
    return jnp.pad(wsa, ((0, 0), (0, 8)))


def kernel(node_features, edge_index, Win, b_in, W0, a0, W1, a1, W2, a2,
           Wpool, b_pool):
    src = jnp.concatenate(
        [edge_index[0], jnp.zeros((EPAD - E,), jnp.int32)])
    dst = jnp.concatenate(
        [edge_index[1], jnp.zeros((EPAD - E,), jnp.int32)])
    zeros_slab = jnp.zeros((SLAB, HID), jnp.float32)
    b2 = b_in.reshape(1, HID)

    x_or_out = node_features
    first = True
    for w, a in ((W0, a0), (W1, a1), (W2, a2)):
        wsa = _build_wsa(w, a)
        if first:
            h, s = _dense0(x_or_out, Win, b2, w, wsa)
        else:
            h, s = _densek(x_or_out, w, wsa)
        stats = _stats_k(src, dst, s)
        x_or_out = _msg_k(src, dst, s, h, stats, zeros_slab)
        first = False

    x, he = _pool(x_or_out, Wpool.reshape(1, HID), b_pool.reshape(1, 1))
    return (he.reshape(HID), x)


# speedup vs baseline: 23.0144x; 1.1312x over previous
import functools

import jax
import jax.numpy as jnp
from jax import lax
from jax.experimental import pallas as pl
from jax.experimental.pallas import tpu as pltpu
from jax.experimental.pallas import tpu_sc as plsc

N = 50000
E = 800000
DIN = 25
HID = 64
NH = 4

NCORES = 2
NSUB = 16
NW = NCORES * NSUB

HALF = N // 2
SLAB = 1568
HPAD = NSUB * SLAB
TAIL = HALF - (NSUB - 1) * SLAB

C = 32
CS = 128
EPT_MSG = E // NSUB
NCH_MSG = 2 * ((EPT_MSG + 2 * C - 1) // (2 * C))
EPT_ST = E // NW
NCH_ST = (EPT_ST + CS - 1) // CS
EPAD = E + 1024

_mesh = plsc.VectorSubcoreMesh(core_axis_name="c", subcore_axis_name="s")


def _dense0_body(x_ref, w1_ref, b_ref, w_ref, wsa_ref, h_ref, s_ref):
    x = jnp.dot(x_ref[...], w1_ref[...], preferred_element_type=jnp.float32)
    x = x + b_ref[...]
    h_ref[...] = jnp.dot(x, w_ref[...], preferred_element_type=jnp.float32)
    s_ref[...] = jnp.dot(x, wsa_ref[...], preferred_element_type=jnp.float32)


def _densek_body(x_ref, w_ref, wsa_ref, h_ref, s_ref):
    v = x_ref[...]
    x = jnp.where(v > 0, v, jnp.exp(v) - 1.0)
    h_ref[...] = jnp.dot(x, w_ref[...], preferred_element_type=jnp.float32)
    s_ref[...] = jnp.dot(x, wsa_ref[...], preferred_element_type=jnp.float32)


_DENSE_OUT = dict(
    out_specs=[
        pl.BlockSpec((2000, NH * HID), lambda i: (i, 0)),
        pl.BlockSpec((2000, 16), lambda i: (i, 0)),
    ],
    out_shape=[
        jax.ShapeDtypeStruct((N, NH * HID), jnp.float32),
        jax.ShapeDtypeStruct((N, 16), jnp.float32),
    ],
)


def _dense0(xin, w1, b2, w, wsa):
    blk = 2000
    return pl.pallas_call(
        _dense0_body,
        grid=(N // blk,),
        in_specs=[
            pl.BlockSpec((blk, DIN), lambda i: (i, 0)),
            pl.BlockSpec((DIN, HID), lambda i: (0, 0)),
            pl.BlockSpec((1, HID), lambda i: (0, 0)),
            pl.BlockSpec((HID, NH * HID), lambda i: (0, 0)),
            pl.BlockSpec((HID, 16), lambda i: (0, 0)),
        ],
        **_DENSE_OUT,
    )(xin, w1, b2, w, wsa)


def _densek(xin, w, wsa):
    blk = 2000
    return pl.pallas_call(
        _densek_body,
        grid=(N // blk,),
        in_specs=[
            pl.BlockSpec((blk, HID), lambda i: (i, 0)),
            pl.BlockSpec((HID, NH * HID), lambda i: (0, 0)),
            pl.BlockSpec((HID, 16), lambda i: (0, 0)),
        ],
        **_DENSE_OUT,
    )(xin, w, wsa)


def _pool_body(o_ref, wp_ref, bp_ref, x_ref, he_ref, acc_ref, m_ref, z_ref):
    i = pl.program_id(0)

    @pl.when(i == 0)
    def _():
        m_ref[0, 0] = -1e30
        z_ref[0, 0] = 0.0
        acc_ref[...] = jnp.zeros_like(acc_ref)

    v = o_ref[...]
    x = jnp.where(v > 0, v, jnp.exp(v) - 1.0)
    x_ref[...] = x
    sc = jnp.sum(x * wp_ref[...], axis=1, keepdims=True) + bp_ref[0, 0]
    bm = jnp.max(sc)
    m_old = m_ref[0, 0]
    m_new = jnp.maximum(m_old, bm)
    scale = jnp.exp(m_old - m_new)
    e = jnp.exp(sc - m_new)
    z_new = z_ref[0, 0] * scale + jnp.sum(e)
    z_ref[0, 0] = z_new
    acc = acc_ref[0:1, :] * scale + jnp.sum(x * e, axis=0, keepdims=True)
    acc_ref[0:1, :] = acc
    m_ref[0, 0] = m_new
    he_ref[...] = acc / jnp.maximum(z_new, 1e-30)


def _pool(out2, wpT, bp):
    blk = 2000
    grid = (N // blk,)
    return pl.pallas_call(
        _pool_body,
        grid=grid,
        in_specs=[
            pl.BlockSpec((blk, HID), lambda i: (i, 0)),
            pl.BlockSpec((1, HID), lambda i: (0, 0)),
            pl.BlockSpec((1, 1), lambda i: (0, 0)),
        ],
        out_specs=[
            pl.BlockSpec((blk, HID), lambda i: (i, 0)),
            pl.BlockSpec((1, HID), lambda i: (0, 0)),
        ],
        out_shape=[
            jax.ShapeDtypeStruct((N, HID), jnp.float32),
            jax.ShapeDtypeStruct((1, HID), jnp.float32),
        ],
        scratch_shapes=[
            pltpu.VMEM((1, HID), jnp.float32),
            pltpu.SMEM((1, 1), jnp.float32),
            pltpu.SMEM((1, 1), jnp.float32),
        ],
        compiler_params=pltpu.CompilerParams(
            dimension_semantics=("arbitrary",)),
    )(out2, wpT, bp)


def _take(x, idx):
    dn = lax.GatherDimensionNumbers(
        offset_dims=(), collapsed_slice_dims=(0,), start_index_map=(0,))
    return lax.gather(x, idx[:, None], dn, (1,),
                      mode=lax.GatherScatterMode.PROMISE_IN_BOUNDS)


@functools.partial(
    pl.kernel,
    out_type=jax.ShapeDtypeStruct((NW, 16), jnp.float32),
    mesh=_mesh,
    scratch_types=[
        pltpu.VMEM((CS,), jnp.int32),
        pltpu.VMEM((CS,), jnp.int32),
        pltpu.VMEM((CS, 16), jnp.float32),
        pltpu.VMEM((CS, 16), jnp.float32),
        pltpu.VMEM((16,), jnp.float32),
        pltpu.SemaphoreType.DMA,
        pltpu.SemaphoreType.DMA,
    ],
    compiler_params=pltpu.CompilerParams(use_tc_tiling_on_sc=False),
)
def _stats_k(src_hbm, dst_hbm, s_hbm, stats_hbm,
             srcv, dstv, ssv, sdv, statv, sem1, sem2):
    cid = lax.axis_index("c")
    sid = lax.axis_index("s")
    tid = sid * NCORES + cid
    base = tid * EPT_ST
    iota = lax.iota(jnp.int32, 16)
    sh4 = jnp.minimum(iota + NH, 15)
    bk4 = jnp.maximum(iota - NH, 0)

    def chunk(ci, carry):
        off = base + ci * CS
        pltpu.sync_copy(src_hbm.at[pl.ds(off, CS)], srcv)
        pltpu.sync_copy(dst_hbm.at[pl.ds(off, CS)], dstv)
        cp1 = pltpu.async_copy(s_hbm.at[srcv], ssv, sem1)
        cp2 = pltpu.async_copy(s_hbm.at[dstv], sdv, sem2)
        cp1.wait()
        cp2.wait()

        def edge(e, ms):
            m, z = ms
            valid = (ci * CS + e) < EPT_ST
            l = ssv[e] + _take(sdv[e], sh4)
            l = jnp.where(l > 0, l, 0.2 * l)
            l = jnp.where(valid, l, -1e30)
            m_new = jnp.maximum(m, l)
            et = jnp.where(valid, jnp.exp(l - m_new), 0.0)
            z = z * jnp.exp(m - m_new) + et
            return (m_new, z)

        return lax.fori_loop(0, CS, edge, carry)

    m, z = lax.fori_loop(
        0, NCH_ST, chunk,
        (jnp.full((16,), -1e30, jnp.float32), jnp.zeros((16,), jnp.float32)))

    v = jnp.where(iota < NH, m, jnp.where(iota < 2 * NH, _take(z, bk4), 0.0))
    statv[...] = v
    pltpu.sync_copy(statv, stats_hbm.at[tid])


@functools.partial(
    pl.kernel,
    out_type=jax.ShapeDtypeStruct((N, HID), jnp.float32),
    mesh=_mesh,
    scratch_types=[
        pltpu.VMEM((2, C), jnp.int32),
        pltpu.VMEM((2, C), jnp.int32),
        pltpu.VMEM((C,), jnp.int32),
        pltpu.VMEM((2, C, 16), jnp.float32),
        pltpu.VMEM((2, C, 16), jnp.float32),
        pltpu.VMEM((2, C, NH * HID), jnp.float32),
        pltpu.VMEM((C, HID), jnp.float32),
        pltpu.VMEM((NW, 16), jnp.float32),
        pltpu.VMEM_SHARED((HPAD, HID), jnp.float32),
        pltpu.SemaphoreType.DMA,
        pltpu.SemaphoreType.DMA,
        pltpu.SemaphoreType.DMA,
        pltpu.SemaphoreType.DMA,
        pltpu.SemaphoreType.DMA,
        pltpu.SemaphoreType.DMA,
    ],
    compiler_params=pltpu.CompilerParams(use_tc_tiling_on_sc=False),
)
def _msg_k(src_hbm, dst_hbm, s_hbm, h_hbm, stats_hbm, zeros_hbm, out_hbm,
           srcv2, dstv2, sidxv, ssv2, sdv2, hhv2, msgv,
           statv, accsh, *sems):
    cid = lax.axis_index("c")
    sid = lax.axis_index("s")
    iota = lax.iota(jnp.int32, 16)
    sh4 = jnp.minimum(iota + NH, 15)

    pltpu.sync_copy(zeros_hbm, accsh.at[pl.ds(sid * SLAB, SLAB)])
    plsc.subcore_barrier()

    pltpu.sync_copy(stats_hbm, statv)
    m16 = jnp.full((16,), -1e30, jnp.float32)
    for t in range(NW):
        m16 = jnp.maximum(m16, statv[t])
    z16 = jnp.zeros((16,), jnp.float32)
    for t in range(NW):
        row = statv[t]
        z16 = z16 + _take(row, sh4) * jnp.exp(row - m16)
    wvec = 1.0 / (NH * z16)

    base = sid * EPT_MSG
    dlo = cid * HALF
    bufs = [(srcv2.at[b], dstv2.at[b], ssv2.at[b], sdv2.at[b], hhv2.at[b],
             sems[3 * b], sems[3 * b + 1], sems[3 * b + 2]) for b in (0, 1)]

    def fire(b, ci):
        srcv, dstv, ssv, sdv, hhv, s1, s2, s3 = bufs[b]
        off = base + ci * C
        pltpu.sync_copy(src_hbm.at[pl.ds(off, C)], srcv)
        pltpu.sync_copy(dst_hbm.at[pl.ds(off, C)], dstv)
        pltpu.async_copy(s_hbm.at[srcv], ssv, s1)
        pltpu.async_copy(s_hbm.at[dstv], sdv, s2)
        pltpu.async_copy(h_hbm.at[srcv], hhv, s3)

    def wait(b):
        srcv, dstv, ssv, sdv, hhv, s1, s2, s3 = bufs[b]
        pltpu.make_async_copy(s_hbm.at[srcv], ssv, s1).wait()
        pltpu.make_async_copy(s_hbm.at[dstv], sdv, s2).wait()
        pltpu.make_async_copy(h_hbm.at[srcv], hhv, s3).wait()

    def process(b, ci):
        srcv, dstv, ssv, sdv, hhv, s1, s2, s3 = bufs[b]
        for g in range(C // 16):
            rows = iota + g * 16
            valid = (rows + ci * C) < EPT_MSG
            dv = dstv[pl.ds(g * 16, 16)]
            dl = dv - dlo
            inhalf = valid & (dl >= 0) & (dl < HALF)
            sidxv[pl.ds(g * 16, 16)] = jnp.where(inhalf, dl, HALF)

        def edge(e, _):
            l = ssv[e] + _take(sdv[e], sh4)
            l = jnp.where(l > 0, l, 0.2 * l)
            w16 = jnp.exp(l - m16) * wvec
            wb = [_take(w16, jnp.full((16,), h, jnp.int32)) for h in range(NH)]
            for vblk in range(HID // 16):
                acc = wb[0] * hhv[e, pl.ds(vblk * 16, 16)]
                for h in range(1, NH):
                    acc = acc + wb[h] * hhv[e, pl.ds(h * HID + vblk * 16, 16)]
                msgv[e, pl.ds(vblk * 16, 16)] = acc
            return 0

        lax.fori_loop(0, C, edge, 0)
        pltpu.sync_copy(msgv, accsh.at[sidxv], add=True)

    fire(0, 0)
    fire(1, 1)

    def pair(p, _):
        ci = p * 2
        wait(0)
        process(0, ci)
        fire(0, ci + 2)
        wait(1)
        process(1, ci + 1)
        fire(1, ci + 3)
        return 0

    lax.fori_loop(0, NCH_MSG // 2, pair, 0)
    wait(0)
    wait(1)
    plsc.subcore_barrier()

    @pl.when(sid < NSUB - 1)
    def _():
        pltpu.sync_copy(accsh.at[pl.ds(sid * SLAB, SLAB)],
                        out_hbm.at[pl.ds(dlo + sid * SLAB, SLAB)])

    @pl.when(sid == NSUB - 1)
    def _():
        pltpu.sync_copy(accsh.at[pl.ds((NSUB - 1) * SLAB, TAIL)],
                        out_hbm.at[pl.ds(dlo + (NSUB - 1) * SLAB, TAIL)])


def _build_wsa(w, a):
    a_src = a[:, :HID]
    a_dst = a[:, HID:]
    eye = jnp.eye(NH, dtype=jnp.float32)
    m1 = (eye[:, None, :] * a_src[:, :, None]).reshape(NH * HID, NH)
    m2 = (eye[:, None, :] * a_dst[:, :, None]).reshape(NH * HID, NH)
    wsa = w @ jnp.concatenate([m1, m2], axis=1)
    return jnp.pad(wsa, ((0, 0), (0, 8)))


def kernel(node_features, edge_index, Win, b_in, W0, a0, W1, a1, W2, a2,
           Wpool, b_pool):
    src = jnp.concatenate(
        [edge_index[0], jnp.zeros((EPAD - E,), jnp.int32)])
    dst = jnp.concatenate(
        [edge_index[1], jnp.zeros((EPAD - E,), jnp.int32)])
    zeros_slab = jnp.zeros((SLAB, HID), jnp.float32)
    b2 = b_in.reshape(1, HID)

    x_or_out = node_features
    first = True
    for w, a in ((W0, a0), (W1, a1), (W2, a2)):
        wsa = _build_wsa(w, a)
        if first:
            h, s = _dense0(x_or_out, Win, b2, w, wsa)
        else:
            h, s = _densek(x_or_out, w, wsa)
        stats = _stats_k(src, dst, s)
        x_or_out = _msg_k(src, dst, s, h, stats, zeros_slab)
        first = False

    x, he = _pool(x_or_out, Wpool.reshape(1, HID), b_pool.reshape(1, 1))
    return (he.reshape(HID), x)


# speedup vs baseline: 38.1673x; 1.6584x over previous
import functools

import jax
import jax.numpy as jnp
import numpy as np
from jax import lax
from jax.experimental import pallas as pl
from jax.experimental.pallas import tpu as pltpu
from jax.experimental.pallas import tpu_sc as plsc

N = 50000
E = 800000
DIN = 25
HID = 64
NH = 4

NCORES = 2
NSUB = 16
NW = NCORES * NSUB

HALF = N // 2
SLAB = 1568
HPAD = NSUB * SLAB
TAIL = HALF - (NSUB - 1) * SLAB

C = 64
CS = 128
EPT_MSG = E // NSUB
NCH_MSG = 2 * ((EPT_MSG + 2 * C - 1) // (2 * C))
EPT_ST = E // NW
NCH_ST = (EPT_ST + CS - 1) // CS
EPAD = E + 1024

_mesh = plsc.VectorSubcoreMesh(core_axis_name="c", subcore_axis_name="s")


def _dense0_body(x_ref, w1_ref, b_ref, w_ref, wsa_ref, h_ref, s_ref):
    x = jnp.dot(x_ref[...], w1_ref[...], preferred_element_type=jnp.float32)
    x = x + b_ref[...]
    h_ref[...] = jnp.dot(
        x, w_ref[...], preferred_element_type=jnp.float32).astype(jnp.bfloat16)
    s_ref[...] = jnp.dot(x, wsa_ref[...], preferred_element_type=jnp.float32)


def _densek_body(x_ref, w_ref, wsa_ref, h_ref, s_ref):
    v = x_ref[...]
    x = jnp.where(v > 0, v, jnp.exp(v) - 1.0)
    h_ref[...] = jnp.dot(
        x, w_ref[...], preferred_element_type=jnp.float32).astype(jnp.bfloat16)
    s_ref[...] = jnp.dot(x, wsa_ref[...], preferred_element_type=jnp.float32)


_DENSE_OUT = dict(
    out_specs=[
        pl.BlockSpec((2000, NH * HID), lambda i: (i, 0)),
        pl.BlockSpec((2000, 16), lambda i: (i, 0)),
    ],
    out_shape=[
        jax.ShapeDtypeStruct((N, NH * HID), jnp.bfloat16),
        jax.ShapeDtypeStruct((N, 16), jnp.float32),
    ],
)


def _dense0(xin, w1, b2, w, wsa):
    blk = 2000
    return pl.pallas_call(
        _dense0_body,
        grid=(N // blk,),
        in_specs=[
            pl.BlockSpec((blk, DIN), lambda i: (i, 0)),
            pl.BlockSpec((DIN, HID), lambda i: (0, 0)),
            pl.BlockSpec((1, HID), lambda i: (0, 0)),
            pl.BlockSpec((HID, NH * HID), lambda i: (0, 0)),
            pl.BlockSpec((HID, 16), lambda i: (0, 0)),
        ],
        **_DENSE_OUT,
    )(xin, w1, b2, w, wsa)


def _densek(xin, w, wsa):
    blk = 2000
    return pl.pallas_call(
        _densek_body,
        grid=(N // blk,),
        in_specs=[
            pl.BlockSpec((blk, HID), lambda i: (i, 0)),
            pl.BlockSpec((HID, NH * HID), lambda i: (0, 0)),
            pl.BlockSpec((HID, 16), lambda i: (0, 0)),
        ],
        **_DENSE_OUT,
    )(xin, w, wsa)


def _pool_body(o_ref, wp_ref, bp_ref, x_ref, he_ref, acc_ref, m_ref, z_ref):
    i = pl.program_id(0)

    @pl.when(i == 0)
    def _():
        m_ref[0, 0] = -1e30
        z_ref[0, 0] = 0.0
        acc_ref[...] = jnp.zeros_like(acc_ref)

    v = o_ref[...]
    x = jnp.where(v > 0, v, jnp.exp(v) - 1.0)
    x_ref[...] = x
    sc = jnp.sum(x * wp_ref[...], axis=1, keepdims=True) + bp_ref[0, 0]
    bm = jnp.max(sc)
    m_old = m_ref[0, 0]
    m_new = jnp.maximum(m_old, bm)
    scale = jnp.exp(m_old - m_new)
    e = jnp.exp(sc - m_new)
    z_new = z_ref[0, 0] * scale + jnp.sum(e)
    z_ref[0, 0] = z_new
    acc = acc_ref[0:1, :] * scale + jnp.sum(x * e, axis=0, keepdims=True)
    acc_ref[0:1, :] = acc
    m_ref[0, 0] = m_new
    he_ref[...] = acc / jnp.maximum(z_new, 1e-30)


def _pool(out2, wpT, bp):
    blk = 2000
    grid = (N // blk,)
    return pl.pallas_call(
        _pool_body,
        grid=grid,
        in_specs=[
            pl.BlockSpec((blk, HID), lambda i: (i, 0)),
            pl.BlockSpec((1, HID), lambda i: (0, 0)),
            pl.BlockSpec((1, 1), lambda i: (0, 0)),
        ],
        out_specs=[
            pl.BlockSpec((blk, HID), lambda i: (i, 0)),
            pl.BlockSpec((1, HID), lambda i: (0, 0)),
        ],
        out_shape=[
            jax.ShapeDtypeStruct((N, HID), jnp.float32),
            jax.ShapeDtypeStruct((1, HID), jnp.float32),
        ],
        scratch_shapes=[
            pltpu.VMEM((1, HID), jnp.float32),
            pltpu.SMEM((1, 1), jnp.float32),
            pltpu.SMEM((1, 1), jnp.float32),
        ],
        compiler_params=pltpu.CompilerParams(
            dimension_semantics=("arbitrary",)),
    )(out2, wpT, bp)


def _take(x, idx):
    dn = lax.GatherDimensionNumbers(
        offset_dims=(), collapsed_slice_dims=(0,), start_index_map=(0,))
    return lax.gather(x, idx[:, None], dn, (1,),
                      mode=lax.GatherScatterMode.PROMISE_IN_BOUNDS)


@functools.partial(
    pl.kernel,
    out_type=jax.ShapeDtypeStruct((NW, 16), jnp.float32),
    mesh=_mesh,
    scratch_types=[
        pltpu.VMEM((CS,), jnp.int32),
        pltpu.VMEM((CS,), jnp.int32),
        pltpu.VMEM((CS, 16), jnp.float32),
        pltpu.VMEM((CS, 16), jnp.float32),
        pltpu.VMEM((16,), jnp.float32),
        pltpu.SemaphoreType.DMA,
        pltpu.SemaphoreType.DMA,
    ],
    compiler_params=pltpu.CompilerParams(use_tc_tiling_on_sc=False),
)
def _stats_k(src_hbm, dst_hbm, s_hbm, stats_hbm,
             srcv, dstv, ssv, sdv, statv, sem1, sem2):
    cid = lax.axis_index("c")
    sid = lax.axis_index("s")
    tid = sid * NCORES + cid
    base = tid * EPT_ST
    iota = lax.iota(jnp.int32, 16)
    sh4 = jnp.minimum(iota + NH, 15)
    bk4 = jnp.maximum(iota - NH, 0)

    def chunk(ci, carry):
        off = base + ci * CS
        pltpu.sync_copy(src_hbm.at[pl.ds(off, CS)], srcv)
        pltpu.sync_copy(dst_hbm.at[pl.ds(off, CS)], dstv)
        cp1 = pltpu.async_copy(s_hbm.at[srcv], ssv, sem1)
        cp2 = pltpu.async_copy(s_hbm.at[dstv], sdv, sem2)
        cp1.wait()
        cp2.wait()

        def edge(e, ms):
            m, z = ms
            valid = (ci * CS + e) < EPT_ST
            l = ssv[e] + _take(sdv[e], sh4)
            l = jnp.where(l > 0, l, 0.2 * l)
            l = jnp.where(valid, l, -1e30)
            m_new = jnp.maximum(m, l)
            et = jnp.where(valid, jnp.exp(l - m_new), 0.0)
            z = z * jnp.exp(m - m_new) + et
            return (m_new, z)

        return lax.fori_loop(0, CS, edge, carry)

    m, z = lax.fori_loop(
        0, NCH_ST, chunk,
        (jnp.full((16,), -1e30, jnp.float32), jnp.zeros((16,), jnp.float32)))

    v = jnp.where(iota < NH, m, jnp.where(iota < 2 * NH, _take(z, bk4), 0.0))
    statv[...] = v
    pltpu.sync_copy(statv, stats_hbm.at[tid])


@functools.partial(
    pl.kernel,
    out_type=jax.ShapeDtypeStruct((N, HID), jnp.float32),
    mesh=_mesh,
    scratch_types=[
        pltpu.VMEM((2, C), jnp.int32),
        pltpu.VMEM((2, C), jnp.int32),
        pltpu.VMEM((C,), jnp.int32),
        pltpu.VMEM((2, C, 16), jnp.float32),
        pltpu.VMEM((2, C, 16), jnp.float32),
        pltpu.VMEM((2, C, NH * HID), jnp.bfloat16),
        pltpu.VMEM((C, HID), jnp.float32),
        pltpu.VMEM((NW, 16), jnp.float32),
        pltpu.VMEM_SHARED((HPAD, HID), jnp.float32),
        pltpu.SemaphoreType.DMA,
        pltpu.SemaphoreType.DMA,
        pltpu.SemaphoreType.DMA,
        pltpu.SemaphoreType.DMA,
        pltpu.SemaphoreType.DMA,
        pltpu.SemaphoreType.DMA,
    ],
    compiler_params=pltpu.CompilerParams(
        use_tc_tiling_on_sc=False, needs_layout_passes=False),
)
def _msg_k(src_hbm, dst_hbm, s_hbm, h_hbm, stats_hbm, zeros_hbm, out_hbm,
           srcv2, dstv2, sidxv, ssv2, sdv2, hhv2, msgv,
           statv, accsh, *sems):
    cid = lax.axis_index("c")
    sid = lax.axis_index("s")
    iota = lax.iota(jnp.int32, 16)
    sh4 = jnp.minimum(iota + NH, 15)

    pltpu.sync_copy(zeros_hbm, accsh.at[pl.ds(sid * SLAB, SLAB)])
    plsc.subcore_barrier()

    pltpu.sync_copy(stats_hbm, statv)
    m16 = jnp.full((16,), -1e30, jnp.float32)
    for t in range(NW):
        m16 = jnp.maximum(m16, statv[t])
    z16 = jnp.zeros((16,), jnp.float32)
    for t in range(NW):
        row = statv[t]
        z16 = z16 + _take(row, sh4) * jnp.exp(row - m16)
    wvec = 1.0 / (NH * z16)

    base = sid * EPT_MSG
    dlo = cid * HALF
    bufs = [(srcv2.at[b], dstv2.at[b], ssv2.at[b], sdv2.at[b], hhv2.at[b],
             sems[3 * b], sems[3 * b + 1], sems[3 * b + 2]) for b in (0, 1)]

    def fire(b, ci):
        srcv, dstv, ssv, sdv, hhv, s1, s2, s3 = bufs[b]
        off = base + ci * C
        pltpu.sync_copy(src_hbm.at[pl.ds(off, C)], srcv)
        pltpu.sync_copy(dst_hbm.at[pl.ds(off, C)], dstv)
        pltpu.async_copy(s_hbm.at[srcv], ssv, s1)
        pltpu.async_copy(s_hbm.at[dstv], sdv, s2)
        pltpu.async_copy(h_hbm.at[srcv], hhv, s3)

    def wait(b):
        srcv, dstv, ssv, sdv, hhv, s1, s2, s3 = bufs[b]
        pltpu.make_async_copy(s_hbm.at[srcv], ssv, s1).wait()
        pltpu.make_async_copy(s_hbm.at[dstv], sdv, s2).wait()
        pltpu.make_async_copy(h_hbm.at[srcv], hhv, s3).wait()

    def process(b, ci):
        srcv, dstv, ssv, sdv, hhv, s1, s2, s3 = bufs[b]
        for g in range(C // 16):
            rows = iota + g * 16
            valid = (rows + ci * C) < EPT_MSG
            dv = dstv[pl.ds(g * 16, 16)]
            dl = dv - dlo
            inhalf = valid & (dl >= 0) & (dl < HALF)
            sidxv[pl.ds(g * 16, 16)] = jnp.where(inhalf, dl, HALF)

        def edge(e, _):
            l = ssv[e] + _take(sdv[e], sh4)
            l = jnp.where(l > 0, l, 0.2 * l)
            w16 = jnp.exp(l - m16) * wvec
            wb = [_take(w16, jnp.full((16,), h, jnp.int32)) for h in range(NH)]
            accs = [None] * 4
            for h in range(NH):
                for j2 in range(2):
                    g = hhv[e, pl.ds(h * HID + j2 * 32, 32)]
                    pa, pb = plsc.unpack(
                        g, format=plsc.PackFormat.INTERLEAVED,
                        preferred_element_type=jnp.float32)
                    for v, pv in ((j2 * 2, pa), (j2 * 2 + 1, pb)):
                        t = wb[h] * pv
                        accs[v] = t if accs[v] is None else accs[v] + t
            for v in range(4):
                msgv[e, pl.ds(v * 16, 16)] = accs[v]
            return 0

        lax.fori_loop(0, C, edge, 0)
        pltpu.sync_copy(msgv, accsh.at[sidxv], add=True)

    fire(0, 0)
    fire(1, 1)

    def pair(p, _):
        ci = p * 2
        wait(0)
        process(0, ci)
        fire(0, ci + 2)
        wait(1)
        process(1, ci + 1)
        fire(1, ci + 3)
        return 0

    lax.fori_loop(0, NCH_MSG // 2, pair, 0)
    wait(0)
    wait(1)
    plsc.subcore_barrier()

    @pl.when(sid < NSUB - 1)
    def _():
        pltpu.sync_copy(accsh.at[pl.ds(sid * SLAB, SLAB)],
                        out_hbm.at[pl.ds(dlo + sid * SLAB, SLAB)])

    @pl.when(sid == NSUB - 1)
    def _():
        pltpu.sync_copy(accsh.at[pl.ds((NSUB - 1) * SLAB, TAIL)],
                        out_hbm.at[pl.ds(dlo + (NSUB - 1) * SLAB, TAIL)])


def _build_wsa(w, a):
    a_src = a[:, :HID]
    a_dst = a[:, HID:]
    eye = jnp.eye(NH, dtype=jnp.float32)
    m1 = (eye[:, None, :] * a_src[:, :, None]).reshape(NH * HID, NH)
    m2 = (eye[:, None, :] * a_dst[:, :, None]).reshape(NH * HID, NH)
    wsa = w @ jnp.concatenate([m1, m2], axis=1)
    return jnp.pad(wsa, ((0, 0), (0, 8)))


_HPERM = np.zeros(NH * HID, np.int32)
for _h in range(NH):
    for _j2 in range(2):
        for _i in range(16):
            _HPERM[_h * 64 + _j2 * 32 + 2 * _i] = _h * 64 + _j2 * 32 + _i
            _HPERM[_h * 64 + _j2 * 32 + 2 * _i + 1] = (
                _h * 64 + _j2 * 32 + 16 + _i)


def kernel(node_features, edge_index, Win, b_in, W0, a0, W1, a1, W2, a2,
           Wpool, b_pool):
    src = jnp.concatenate(
        [edge_index[0], jnp.zeros((EPAD - E,), jnp.int32)])
    dst = jnp.concatenate(
        [edge_index[1], jnp.zeros((EPAD - E,), jnp.int32)])
    zeros_slab = jnp.zeros((SLAB, HID), jnp.float32)
    b2 = b_in.reshape(1, HID)

    x_or_out = node_features
    first = True
    for w, a in ((W0, a0), (W1, a1), (W2, a2)):
        wsa = _build_wsa(w, a)
        if first:
            h, s = _dense0(x_or_out, Win, b2, w[:, _HPERM], wsa)
        else:
            h, s = _densek(x_or_out, w[:, _HPERM], wsa)
        stats = _stats_k(src, dst, s)
        x_or_out = _msg_k(src, dst, s, h, stats, zeros_slab)
        first = False

    x, he = _pool(x_or_out, Wpool.reshape(1, HID), b_pool.reshape(1, 1))
    return (he.reshape(HID), x)
